# Initial kernel scaffold; baseline (speedup 1.0000x reference)
import jax
import jax.numpy as jnp
from jax.experimental import pallas as pl


def kernel(x, edge_index, W1, b1, W2, b2, W3, b3, W4, b4, W5, b5, W6, b6, W7, b7, Wc, bc):
    raise NotImplementedError("write your pallas kernel here")


# speedup vs baseline: 42.3297x; 42.3297x over previous
import functools

import jax
import jax.numpy as jnp
from jax import lax
from jax.experimental import pallas as pl
from jax.experimental.pallas import tpu as pltpu
from jax.experimental.pallas import tpu_sc as plsc

N = 100000
E = 1600000
NC, NS, LANES = 2, 16, 16
NW = NC * NS
NPAD = 100352
CNODE = NPAD // NW
NV = CNODE // 16
SPCH = NPAD // NS
EPAD = 1605632
ETILE = EPAD // NW
EROWS = ETILE // 128
BM = 800
DIMS = [3, 6, 3, 3, 3, 2, 2]


def _mesh():
    return plsc.VectorSubcoreMesh(core_axis_name="c", subcore_axis_name="s")


def _rsqrt16(v):
    i = lax.bitcast_convert_type(v, jnp.int32)
    i = jnp.int32(0x5F3759DF) - lax.shift_right_logical(i, 1)
    y = lax.bitcast_convert_type(i, jnp.float32)
    for _ in range(3):
        y = y * (1.5 - 0.5 * v * y * y)
    return y


def _tanh16(y):
    e = jnp.exp(2.0 * y)
    return 1.0 - 2.0 / (e + 1.0)


def _zero_fill(buf, nwords):
    z = jnp.zeros((16,), jnp.float32)

    def f(k, c):
        buf[pl.ds(k * 16, 16)] = z
        return c

    lax.fori_loop(0, nwords // 16, f, 0)


def _wid():
    return lax.axis_index("c") * NS + lax.axis_index("s")


def _xw_body(x_ref, w_ref, o_ref):
    o_ref[...] = jnp.dot(x_ref[...], w_ref[...],
                         preferred_element_type=jnp.float32)


@jax.jit
def _xw1(x, w1p):
    nblk = 126
    return pl.pallas_call(
        _xw_body,
        grid=(nblk,),
        in_specs=[
            pl.BlockSpec((BM, 128), lambda i: (jnp.minimum(i, 124), 0)),
            pl.BlockSpec((128, 8), lambda i: (0, 0)),
        ],
        out_specs=pl.BlockSpec((BM, 8), lambda i: (i, 0)),
        out_shape=jax.ShapeDtypeStruct((nblk * BM, 8), jnp.float32),
    )(x, w1p)


def _make_deg():
    krows = 8
    nwin = EROWS // krows

    def body(dst2d, degp, dstv, ones, zbuf, degsp, ssem):
        cid = lax.axis_index("c")
        sid = lax.axis_index("s")
        wid = cid * NS + sid
        o = jnp.ones((16,), jnp.float32)
        for k in range(8):
            ones[pl.ds(k * 16, 16)] = o
        _zero_fill(zbuf, SPCH)
        pltpu.sync_copy(zbuf, degsp.at[pl.ds(sid * SPCH, SPCH)])
        plsc.subcore_barrier()

        def win(w, c):
            rbase = wid * EROWS + w * krows
            pltpu.sync_copy(dst2d.at[pl.ds(rbase, krows)], dstv)
            descs = [
                pltpu.async_copy(ones, degsp.at[dstv.at[j]], ssem, add=True)
                for j in range(krows)
            ]
            for d in descs:
                d.wait()
            return c

        lax.fori_loop(0, nwin, win, 0)
        plsc.subcore_barrier()
        pltpu.sync_copy(degsp.at[pl.ds(sid * SPCH, SPCH)],
                        degp.at[pl.ds(cid * NPAD + sid * SPCH, SPCH)])

    return pl.kernel(
        body,
        out_type=jax.ShapeDtypeStruct((NC * NPAD,), jnp.float32),
        mesh=_mesh(),
        compiler_params=pltpu.CompilerParams(use_tc_tiling_on_sc=False, needs_layout_passes=False),
        scratch_types=[
            pltpu.VMEM((krows, 128), jnp.int32),
            pltpu.VMEM((128,), jnp.float32),
            pltpu.VMEM((SPCH,), jnp.float32),
            pltpu.VMEM_SHARED((NPAD,), jnp.float32),
            pltpu.SemaphoreType.DMA,
        ],
    )


def _make_disg1():
    d1 = DIMS[0]
    nch = 49
    chw = 64

    def body(degp, h1flat, dis_out, g1, d0v, d1v, disv, idxv, hv, gv, gsem):
        wid = _wid()
        base = wid * CNODE
        pltpu.sync_copy(degp.at[pl.ds(base, CNODE)], d0v)
        pltpu.sync_copy(degp.at[pl.ds(NPAD + base, CNODE)], d1v)

        def f(v, c):
            sl = pl.ds(v * 16, 16)
            deg = d0v[sl] + d1v[sl] + 1.0
            disv[sl] = _rsqrt16(deg)
            return c

        lax.fori_loop(0, NV, f, 0)
        pltpu.sync_copy(disv, dis_out.at[pl.ds(base, CNODE)])
        iota = lax.broadcasted_iota(jnp.int32, (16,), 0)
        for i in range(d1):
            def fi(v, c):
                sl = pl.ds(v * 16, 16)
                idxv[sl] = (base + v * 16 + iota) * 8 + i
                return c

            lax.fori_loop(0, NV, fi, 0)

            def fc(ch, c):
                sl = pl.ds(ch * chw, chw)
                pltpu.async_copy(h1flat.at[idxv.at[sl]], hv.at[sl],
                                 gsem).wait()
                return c

            lax.fori_loop(0, nch, fc, 0)

            def fg(v, c):
                sl = pl.ds(v * 16, 16)
                gv[sl] = disv[sl] * hv[sl]
                return c

            lax.fori_loop(0, NV, fg, 0)
            pltpu.sync_copy(gv, g1.at[pl.ds(i * NPAD + base, CNODE)])

    return pl.kernel(
        body,
        out_type=(
            jax.ShapeDtypeStruct((NPAD,), jnp.float32),
            jax.ShapeDtypeStruct((d1 * NPAD,), jnp.float32),
        ),
        mesh=_mesh(),
        compiler_params=pltpu.CompilerParams(use_tc_tiling_on_sc=False, needs_layout_passes=False),
        scratch_types=[
            pltpu.VMEM((CNODE,), jnp.float32),
            pltpu.VMEM((CNODE,), jnp.float32),
            pltpu.VMEM((CNODE,), jnp.float32),
            pltpu.VMEM((CNODE,), jnp.int32),
            pltpu.VMEM((CNODE,), jnp.float32),
            pltpu.VMEM((CNODE,), jnp.float32),
            pltpu.SemaphoreType.DMA,
        ],
    )


@functools.lru_cache(maxsize=None)
def _make_edge(d):
    krows = 8
    nwin = EROWS // krows

    def body(src2d, dst2d, g_hbm, accp, srcv, dstv, val, zbuf, *rest):
        gsp = rest[:d]
        accsp = rest[d:2 * d]
        gsem, ssem = rest[2 * d], rest[2 * d + 1]
        cid = lax.axis_index("c")
        sid = lax.axis_index("s")
        wid = cid * NS + sid
        ssl = pl.ds(sid * SPCH, SPCH)
        for i in range(d):
            pltpu.sync_copy(g_hbm.at[pl.ds(i * NPAD + sid * SPCH, SPCH)],
                            gsp[i].at[ssl])
        _zero_fill(zbuf, SPCH)
        for i in range(d):
            pltpu.sync_copy(zbuf, accsp[i].at[ssl])
        plsc.subcore_barrier()

        def win(w, c):
            rbase = wid * EROWS + w * krows
            pltpu.sync_copy(src2d.at[pl.ds(rbase, krows)], srcv)
            pltpu.sync_copy(dst2d.at[pl.ds(rbase, krows)], dstv)
            descs = []
            for i in range(d):
                for j in range(krows):
                    descs.append(pltpu.async_copy(
                        gsp[i].at[srcv.at[j]], val.at[i, j], gsem))
            for dd in descs:
                dd.wait()
            descs = []
            for i in range(d):
                for j in range(krows):
                    descs.append(pltpu.async_copy(
                        val.at[i, j], accsp[i].at[dstv.at[j]], ssem,
                        add=True))
            for dd in descs:
                dd.wait()
            return c

        lax.fori_loop(0, nwin, win, 0)
        plsc.subcore_barrier()
        for i in range(d):
            pltpu.sync_copy(
                accsp[i].at[ssl],
                accp.at[pl.ds((cid * d + i) * NPAD + sid * SPCH, SPCH)])

    scratch = [
        pltpu.VMEM((krows, 128), jnp.int32),
        pltpu.VMEM((krows, 128), jnp.int32),
        pltpu.VMEM((d, krows, 128), jnp.float32),
        pltpu.VMEM((SPCH,), jnp.float32),
    ]
    scratch += [pltpu.VMEM_SHARED((NPAD,), jnp.float32) for _ in range(d)]
    scratch += [pltpu.VMEM_SHARED((NPAD,), jnp.float32) for _ in range(d)]
    scratch += [pltpu.SemaphoreType.DMA, pltpu.SemaphoreType.DMA]

    return pl.kernel(
        body,
        out_type=jax.ShapeDtypeStruct((NC * d * NPAD,), jnp.float32),
        mesh=_mesh(),
        compiler_params=pltpu.CompilerParams(use_tc_tiling_on_sc=False, needs_layout_passes=False),
        scratch_types=scratch,
    )


@functools.lru_cache(maxsize=None)
def _make_node(d_in, d_out, act):
    def body(accp, g_hbm, dis_hbm, wn, bn, gnext,
             a0, a1, gv, disv, wv, bv, gn):
        wid = _wid()
        base = wid * CNODE
        nsl = pl.ds(base, CNODE)
        pltpu.sync_copy(dis_hbm.at[nsl], disv)
        pltpu.sync_copy(wn, wv)
        pltpu.sync_copy(bn, bv)
        for i in range(d_in):
            pltpu.sync_copy(accp.at[pl.ds(i * NPAD + base, CNODE)], a0.at[i])
            pltpu.sync_copy(accp.at[pl.ds((d_in + i) * NPAD + base, CNODE)],
                            a1.at[i])
            pltpu.sync_copy(g_hbm.at[pl.ds(i * NPAD + base, CNODE)],
                            gv.at[i])
        wch = [wv[pl.ds(16 * c, 16)] for c in range(4)]
        bvec = bv[pl.ds(0, 16)]
        wb = [[jnp.full((16,), wch[(j * 8 + k) // 16][(j * 8 + k) % 16],
                        jnp.float32)
               for k in range(d_out)] for j in range(d_in)]
        bb = [jnp.full((16,), bvec[i], jnp.float32) for i in range(d_in)]

        def f(v, c):
            sl = pl.ds(v * 16, 16)
            dd = disv[sl]
            ts = []
            for i in range(d_in):
                y = dd * (a0[i, sl] + a1[i, sl] + gv[i, sl]) + bb[i]
                ts.append(_tanh16(y) if act else y)
            for k in range(d_out):
                acc = ts[0] * wb[0][k]
                for j in range(1, d_in):
                    acc = acc + ts[j] * wb[j][k]
                gn[k, sl] = dd * acc
            return c

        lax.fori_loop(0, NV, f, 0)
        for k in range(d_out):
            pltpu.sync_copy(gn.at[k], gnext.at[pl.ds(k * NPAD + base, CNODE)])

    return pl.kernel(
        body,
        out_type=jax.ShapeDtypeStruct((d_out * NPAD,), jnp.float32),
        mesh=_mesh(),
        compiler_params=pltpu.CompilerParams(use_tc_tiling_on_sc=False, needs_layout_passes=False),
        scratch_types=[
            pltpu.VMEM((d_in, CNODE), jnp.float32),
            pltpu.VMEM((d_in, CNODE), jnp.float32),
            pltpu.VMEM((d_in, CNODE), jnp.float32),
            pltpu.VMEM((CNODE,), jnp.float32),
            pltpu.VMEM((64,), jnp.float32),
            pltpu.VMEM((16,), jnp.float32),
            pltpu.VMEM((d_out, CNODE), jnp.float32),
        ],
    )


def _make_final():
    d_in = DIMS[6]

    def body(accp, g_hbm, dis_hbm, wc, bn, bc, out_f, h7_f,
             a0, a1, gv, disv, wv, bv, bcv, ov, hv):
        wid = _wid()
        base = wid * CNODE
        nsl = pl.ds(base, CNODE)
        pltpu.sync_copy(dis_hbm.at[nsl], disv)
        pltpu.sync_copy(wc, wv)
        pltpu.sync_copy(bn, bv)
        pltpu.sync_copy(bc, bcv)
        for i in range(d_in):
            pltpu.sync_copy(accp.at[pl.ds(i * NPAD + base, CNODE)], a0.at[i])
            pltpu.sync_copy(accp.at[pl.ds((d_in + i) * NPAD + base, CNODE)],
                            a1.at[i])
            pltpu.sync_copy(g_hbm.at[pl.ds(i * NPAD + base, CNODE)],
                            gv.at[i])
        wch = [wv[pl.ds(16 * c, 16)] for c in range(4)]
        bvec = bv[pl.ds(0, 16)]
        bcvec = bcv[pl.ds(0, 16)]
        wb = [[jnp.full((16,), wch[(j * 8 + k) // 16][(j * 8 + k) % 16],
                        jnp.float32) for k in range(2)]
              for j in range(d_in)]
        bb = [jnp.full((16,), bvec[i], jnp.float32) for i in range(d_in)]
        cb = [jnp.full((16,), bcvec[k], jnp.float32) for k in range(2)]
        iota = lax.broadcasted_iota(jnp.int32, (16,), 0)

        def f(v, c):
            sl = pl.ds(v * 16, 16)
            dd = disv[sl]
            ts = []
            for i in range(d_in):
                ts.append(dd * (a0[i, sl] + a1[i, sl] + gv[i, sl]) + bb[i])
            idx0 = v * 32 + iota * 2
            for k in range(2):
                ok = ts[0] * wb[0][k]
                for j in range(1, d_in):
                    ok = ok + ts[j] * wb[j][k]
                ok = ok + cb[k]
                plsc.store_scatter(hv, [idx0 + k], ts[k])
                plsc.store_scatter(ov, [idx0 + k], ok)
            return c

        lax.fori_loop(0, NV, f, 0)
        osl = pl.ds(base * 2, 2 * CNODE)
        pltpu.sync_copy(ov, out_f.at[osl])
        pltpu.sync_copy(hv, h7_f.at[osl])

    return pl.kernel(
        body,
        out_type=(
            jax.ShapeDtypeStruct((NPAD * 2,), jnp.float32),
            jax.ShapeDtypeStruct((NPAD * 2,), jnp.float32),
        ),
        mesh=_mesh(),
        compiler_params=pltpu.CompilerParams(use_tc_tiling_on_sc=False, needs_layout_passes=False),
        scratch_types=[
            pltpu.VMEM((d_in, CNODE), jnp.float32),
            pltpu.VMEM((d_in, CNODE), jnp.float32),
            pltpu.VMEM((d_in, CNODE), jnp.float32),
            pltpu.VMEM((CNODE,), jnp.float32),
            pltpu.VMEM((64,), jnp.float32),
            pltpu.VMEM((16,), jnp.float32),
            pltpu.VMEM((16,), jnp.float32),
            pltpu.VMEM((2 * CNODE,), jnp.float32),
            pltpu.VMEM((2 * CNODE,), jnp.float32),
        ],
    )


def _pad_w(w):
    out = jnp.zeros((8, 8), jnp.float32)
    return out.at[: w.shape[0], : w.shape[1]].set(w).reshape(-1)


def _pad_b(b):
    return jnp.zeros((16,), jnp.float32).at[: b.shape[0]].set(b)


def kernel(x, edge_index, W1, b1, W2, b2, W3, b3, W4, b4, W5, b5, W6, b6,
           W7, b7, Wc, bc):
    src = edge_index[0]
    dst = edge_index[1]
    npadidx = (N + (jnp.arange(EPAD - E, dtype=jnp.int32) % (NPAD - N)))
    srcp = jnp.concatenate([src, npadidx])
    dstp = jnp.concatenate([dst, npadidx])
    src2d = srcp.reshape(EPAD // 128, 128)
    dst2d = dstp.reshape(EPAD // 128, 128)

    w1p = jnp.pad(W1, ((0, 0), (0, 8 - W1.shape[1])))
    h1full = _xw1(x, w1p)
    h1flat = h1full[:NPAD].reshape(-1)

    degp = _make_deg()(dst2d)
    dis, g = _make_disg1()(degp, h1flat)

    ws = [W2, W3, W4, W5, W6, W7]
    bs = [b1, b2, b3, b4, b5, b6]
    for l in range(6):
        d_in, d_out = DIMS[l], DIMS[l + 1]
        accp = _make_edge(d_in)(src2d, dst2d, g)
        g = _make_node(d_in, d_out, True)(accp, g, dis, _pad_w(ws[l]),
                                          _pad_b(bs[l]))
    accp = _make_edge(DIMS[6])(src2d, dst2d, g)
    out_f, h7_f = _make_final()(accp, g, dis, _pad_w(Wc), _pad_b(b7),
                                _pad_b(bc))
    out = out_f.reshape(NPAD, 2)[:N]
    h7 = h7_f.reshape(NPAD, 2)[:N]
    return (out, h7)


# speedup vs baseline: 49.9242x; 1.1794x over previous
import functools

import jax
import jax.numpy as jnp
from jax import lax
from jax.experimental import pallas as pl
from jax.experimental.pallas import tpu as pltpu
from jax.experimental.pallas import tpu_sc as plsc

N = 100000
E = 1600000
NC, NS, LANES = 2, 16, 16
NW = NC * NS
NPAD = 100352
CNODE = NPAD // NW
NV = CNODE // 16
SPCH = NPAD // NS
EPAD = 1605632
ETILE = EPAD // NW
EROWS = ETILE // 128
BM = 800
DIMS = [3, 6, 3, 3, 3, 2, 2]


def _mesh():
    return plsc.VectorSubcoreMesh(core_axis_name="c", subcore_axis_name="s")


def _rsqrt16(v):
    i = lax.bitcast_convert_type(v, jnp.int32)
    i = jnp.int32(0x5F3759DF) - lax.shift_right_logical(i, 1)
    y = lax.bitcast_convert_type(i, jnp.float32)
    for _ in range(3):
        y = y * (1.5 - 0.5 * v * y * y)
    return y


def _tanh16(y):
    e = jnp.exp(2.0 * y)
    return 1.0 - 2.0 / (e + 1.0)


def _zero_fill(buf, nwords):
    z = jnp.zeros((16,), jnp.float32)

    def f(k, c):
        buf[pl.ds(k * 16, 16)] = z
        return c

    lax.fori_loop(0, nwords // 16, f, 0)


def _wid():
    return lax.axis_index("c") * NS + lax.axis_index("s")


def _xw_body(x_ref, w_ref, o_ref):
    o_ref[...] = jnp.dot(x_ref[...], w_ref[...],
                         preferred_element_type=jnp.float32)


@jax.jit
def _xw1(x, w1p):
    nblk = 126
    return pl.pallas_call(
        _xw_body,
        grid=(nblk,),
        in_specs=[
            pl.BlockSpec((BM, 128), lambda i: (jnp.minimum(i, 124), 0)),
            pl.BlockSpec((128, 8), lambda i: (0, 0)),
        ],
        out_specs=pl.BlockSpec((BM, 8), lambda i: (i, 0)),
        out_shape=jax.ShapeDtypeStruct((nblk * BM, 8), jnp.float32),
    )(x, w1p)


KR = 14
TOTROWS = EPAD // 128


def _stage(idx3, buf, sem, rbase):
    rr = jnp.minimum(rbase, TOTROWS - KR)
    return pltpu.async_copy(idx3.at[pl.ds(rr, KR)], buf, sem)


def _drain_stage(idx3, buf, sem):
    pltpu.make_async_copy(idx3.at[pl.ds(0, KR)], buf, sem).wait()


def _make_deg():
    nwin = EROWS // KR

    def body(idx3, degp, iA, iB, ones, zbuf, degsp, stA, stB, ssem):
        cid = lax.axis_index("c")
        sid = lax.axis_index("s")
        wid = cid * NS + sid
        o = jnp.ones((16,), jnp.float32)
        for k in range(8):
            ones[pl.ds(k * 16, 16)] = o
        _zero_fill(zbuf, SPCH)
        pltpu.sync_copy(zbuf, degsp.at[pl.ds(sid * SPCH, SPCH)])
        plsc.subcore_barrier()
        rb = wid * EROWS
        _stage(idx3, iA, stA, rb)
        _stage(idx3, iB, stB, rb + KR)

        def win(w, c):
            r0 = rb + w * 2 * KR
            for buf, sem, off in ((iA, stA, 0), (iB, stB, KR)):
                _drain_stage(idx3, buf, sem)
                descs = [
                    pltpu.async_copy(ones, degsp.at[buf.at[j, 1]], ssem,
                                     add=True)
                    for j in range(KR)
                ]
                for dd in descs:
                    dd.wait()
                _stage(idx3, buf, sem, r0 + off + 2 * KR)
            return c

        lax.fori_loop(0, nwin // 2, win, 0)
        _drain_stage(idx3, iA, stA)
        _drain_stage(idx3, iB, stB)
        plsc.subcore_barrier()
        pltpu.sync_copy(degsp.at[pl.ds(sid * SPCH, SPCH)],
                        degp.at[pl.ds(cid * NPAD + sid * SPCH, SPCH)])

    return pl.kernel(
        body,
        out_type=jax.ShapeDtypeStruct((NC * NPAD,), jnp.float32),
        mesh=_mesh(),
        compiler_params=pltpu.CompilerParams(use_tc_tiling_on_sc=False, needs_layout_passes=False),
        scratch_types=[
            pltpu.VMEM((KR, 2, 128), jnp.int32),
            pltpu.VMEM((KR, 2, 128), jnp.int32),
            pltpu.VMEM((128,), jnp.float32),
            pltpu.VMEM((SPCH,), jnp.float32),
            pltpu.VMEM_SHARED((NPAD,), jnp.float32),
            pltpu.SemaphoreType.DMA,
            pltpu.SemaphoreType.DMA,
            pltpu.SemaphoreType.DMA,
        ],
    )


def _make_disg1():
    d1 = DIMS[0]

    def body(degp, h1flat, dis_out, g1, d0v, d1v, disv, idxv, hv, gv, gsem):
        wid = _wid()
        base = wid * CNODE
        pltpu.sync_copy(degp.at[pl.ds(base, CNODE)], d0v)
        pltpu.sync_copy(degp.at[pl.ds(NPAD + base, CNODE)], d1v)

        def f(v, c):
            sl = pl.ds(v * 16, 16)
            deg = d0v[sl] + d1v[sl] + 1.0
            disv[sl] = _rsqrt16(deg)
            return c

        lax.fori_loop(0, NV, f, 0)
        pltpu.sync_copy(disv, dis_out.at[pl.ds(base, CNODE)])
        iota = lax.broadcasted_iota(jnp.int32, (16,), 0)
        for i in range(d1):
            def fi(v, c):
                sl = pl.ds(v * 16, 16)
                idxv[sl] = (base + v * 16 + iota) * 8 + i
                return c

            lax.fori_loop(0, NV, fi, 0)
            pltpu.async_copy(h1flat.at[idxv], hv, gsem).wait()

            def fg(v, c):
                sl = pl.ds(v * 16, 16)
                gv[sl] = disv[sl] * hv[sl]
                return c

            lax.fori_loop(0, NV, fg, 0)
            pltpu.sync_copy(gv, g1.at[pl.ds(i * NPAD + base, CNODE)])

    return pl.kernel(
        body,
        out_type=(
            jax.ShapeDtypeStruct((NPAD,), jnp.float32),
            jax.ShapeDtypeStruct((d1 * NPAD,), jnp.float32),
        ),
        mesh=_mesh(),
        compiler_params=pltpu.CompilerParams(use_tc_tiling_on_sc=False, needs_layout_passes=False),
        scratch_types=[
            pltpu.VMEM((CNODE,), jnp.float32),
            pltpu.VMEM((CNODE,), jnp.float32),
            pltpu.VMEM((CNODE,), jnp.float32),
            pltpu.VMEM((CNODE,), jnp.int32),
            pltpu.VMEM((CNODE,), jnp.float32),
            pltpu.VMEM((CNODE,), jnp.float32),
            pltpu.SemaphoreType.DMA,
        ],
    )


@functools.lru_cache(maxsize=None)
def _make_edge(d):
    nwin = EROWS // KR

    def body(idx3, g_hbm, accp, iA, iB, val, zbuf, *rest):
        gsp = rest[:d]
        accsp = rest[d:2 * d]
        stA, stB = rest[2 * d], rest[2 * d + 1]
        gsem, ssem = rest[2 * d + 2], rest[2 * d + 3]
        cid = lax.axis_index("c")
        sid = lax.axis_index("s")
        wid = cid * NS + sid
        ssl = pl.ds(sid * SPCH, SPCH)
        rb = wid * EROWS
        _stage(idx3, iA, stA, rb)
        _stage(idx3, iB, stB, rb + KR)
        for i in range(d):
            pltpu.sync_copy(g_hbm.at[pl.ds(i * NPAD + sid * SPCH, SPCH)],
                            gsp[i].at[ssl])
        _zero_fill(zbuf, SPCH)
        for i in range(d):
            pltpu.sync_copy(zbuf, accsp[i].at[ssl])
        plsc.subcore_barrier()

        def win(w, c):
            r0 = rb + w * 2 * KR
            for buf, sem, off in ((iA, stA, 0), (iB, stB, KR)):
                _drain_stage(idx3, buf, sem)
                descs = []
                for i in range(d):
                    for j in range(KR):
                        descs.append(pltpu.async_copy(
                            gsp[i].at[buf.at[j, 0]], val.at[i, j], gsem))
                for dd in descs:
                    dd.wait()
                descs = []
                for i in range(d):
                    for j in range(KR):
                        descs.append(pltpu.async_copy(
                            val.at[i, j], accsp[i].at[buf.at[j, 1]], ssem,
                            add=True))
                for dd in descs:
                    dd.wait()
                _stage(idx3, buf, sem, r0 + off + 2 * KR)
            return c

        lax.fori_loop(0, nwin // 2, win, 0)
        _drain_stage(idx3, iA, stA)
        _drain_stage(idx3, iB, stB)
        plsc.subcore_barrier()
        for i in range(d):
            pltpu.sync_copy(
                accsp[i].at[ssl],
                accp.at[pl.ds((cid * d + i) * NPAD + sid * SPCH, SPCH)])

    scratch = [
        pltpu.VMEM((KR, 2, 128), jnp.int32),
        pltpu.VMEM((KR, 2, 128), jnp.int32),
        pltpu.VMEM((d, KR, 128), jnp.float32),
        pltpu.VMEM((SPCH,), jnp.float32),
    ]
    scratch += [pltpu.VMEM_SHARED((NPAD,), jnp.float32) for _ in range(d)]
    scratch += [pltpu.VMEM_SHARED((NPAD,), jnp.float32) for _ in range(d)]
    scratch += [pltpu.SemaphoreType.DMA, pltpu.SemaphoreType.DMA,
                pltpu.SemaphoreType.DMA, pltpu.SemaphoreType.DMA]

    return pl.kernel(
        body,
        out_type=jax.ShapeDtypeStruct((NC * d * NPAD,), jnp.float32),
        mesh=_mesh(),
        compiler_params=pltpu.CompilerParams(use_tc_tiling_on_sc=False, needs_layout_passes=False),
        scratch_types=scratch,
    )


@functools.lru_cache(maxsize=None)
def _make_node(d_in, d_out, act):
    def body(accp, g_hbm, dis_hbm, wn, bn, gnext,
             a0, a1, gv, disv, wv, bv, gn):
        wid = _wid()
        base = wid * CNODE
        nsl = pl.ds(base, CNODE)
        pltpu.sync_copy(dis_hbm.at[nsl], disv)
        pltpu.sync_copy(wn, wv)
        pltpu.sync_copy(bn, bv)
        for i in range(d_in):
            pltpu.sync_copy(accp.at[pl.ds(i * NPAD + base, CNODE)], a0.at[i])
            pltpu.sync_copy(accp.at[pl.ds((d_in + i) * NPAD + base, CNODE)],
                            a1.at[i])
            pltpu.sync_copy(g_hbm.at[pl.ds(i * NPAD + base, CNODE)],
                            gv.at[i])
        wch = [wv[pl.ds(16 * c, 16)] for c in range(4)]
        bvec = bv[pl.ds(0, 16)]
        wb = [[jnp.full((16,), wch[(j * 8 + k) // 16][(j * 8 + k) % 16],
                        jnp.float32)
               for k in range(d_out)] for j in range(d_in)]
        bb = [jnp.full((16,), bvec[i], jnp.float32) for i in range(d_in)]

        def f(v, c):
            sl = pl.ds(v * 16, 16)
            dd = disv[sl]
            ts = []
            for i in range(d_in):
                y = dd * (a0[i, sl] + a1[i, sl] + gv[i, sl]) + bb[i]
                ts.append(_tanh16(y) if act else y)
            for k in range(d_out):
                acc = ts[0] * wb[0][k]
                for j in range(1, d_in):
                    acc = acc + ts[j] * wb[j][k]
                gn[k, sl] = dd * acc
            return c

        lax.fori_loop(0, NV, f, 0)
        for k in range(d_out):
            pltpu.sync_copy(gn.at[k], gnext.at[pl.ds(k * NPAD + base, CNODE)])

    return pl.kernel(
        body,
        out_type=jax.ShapeDtypeStruct((d_out * NPAD,), jnp.float32),
        mesh=_mesh(),
        compiler_params=pltpu.CompilerParams(use_tc_tiling_on_sc=False, needs_layout_passes=False),
        scratch_types=[
            pltpu.VMEM((d_in, CNODE), jnp.float32),
            pltpu.VMEM((d_in, CNODE), jnp.float32),
            pltpu.VMEM((d_in, CNODE), jnp.float32),
            pltpu.VMEM((CNODE,), jnp.float32),
            pltpu.VMEM((64,), jnp.float32),
            pltpu.VMEM((16,), jnp.float32),
            pltpu.VMEM((d_out, CNODE), jnp.float32),
        ],
    )


def _make_final():
    d_in = DIMS[6]

    def body(accp, g_hbm, dis_hbm, wc, bn, bc, out_f, h7_f,
             a0, a1, gv, disv, wv, bv, bcv, ov, hv):
        wid = _wid()
        base = wid * CNODE
        nsl = pl.ds(base, CNODE)
        pltpu.sync_copy(dis_hbm.at[nsl], disv)
        pltpu.sync_copy(wc, wv)
        pltpu.sync_copy(bn, bv)
        pltpu.sync_copy(bc, bcv)
        for i in range(d_in):
            pltpu.sync_copy(accp.at[pl.ds(i * NPAD + base, CNODE)], a0.at[i])
            pltpu.sync_copy(accp.at[pl.ds((d_in + i) * NPAD + base, CNODE)],
                            a1.at[i])
            pltpu.sync_copy(g_hbm.at[pl.ds(i * NPAD + base, CNODE)],
                            gv.at[i])
        wch = [wv[pl.ds(16 * c, 16)] for c in range(4)]
        bvec = bv[pl.ds(0, 16)]
        bcvec = bcv[pl.ds(0, 16)]
        wb = [[jnp.full((16,), wch[(j * 8 + k) // 16][(j * 8 + k) % 16],
                        jnp.float32) for k in range(2)]
              for j in range(d_in)]
        bb = [jnp.full((16,), bvec[i], jnp.float32) for i in range(d_in)]
        cb = [jnp.full((16,), bcvec[k], jnp.float32) for k in range(2)]
        iota = lax.broadcasted_iota(jnp.int32, (16,), 0)

        def f(v, c):
            sl = pl.ds(v * 16, 16)
            dd = disv[sl]
            ts = []
            for i in range(d_in):
                ts.append(dd * (a0[i, sl] + a1[i, sl] + gv[i, sl]) + bb[i])
            idx0 = v * 32 + iota * 2
            for k in range(2):
                ok = ts[0] * wb[0][k]
                for j in range(1, d_in):
                    ok = ok + ts[j] * wb[j][k]
                ok = ok + cb[k]
                plsc.store_scatter(hv, [idx0 + k], ts[k])
                plsc.store_scatter(ov, [idx0 + k], ok)
            return c

        lax.fori_loop(0, NV, f, 0)
        osl = pl.ds(base * 2, 2 * CNODE)
        pltpu.sync_copy(ov, out_f.at[osl])
        pltpu.sync_copy(hv, h7_f.at[osl])

    return pl.kernel(
        body,
        out_type=(
            jax.ShapeDtypeStruct((NPAD * 2,), jnp.float32),
            jax.ShapeDtypeStruct((NPAD * 2,), jnp.float32),
        ),
        mesh=_mesh(),
        compiler_params=pltpu.CompilerParams(use_tc_tiling_on_sc=False, needs_layout_passes=False),
        scratch_types=[
            pltpu.VMEM((d_in, CNODE), jnp.float32),
            pltpu.VMEM((d_in, CNODE), jnp.float32),
            pltpu.VMEM((d_in, CNODE), jnp.float32),
            pltpu.VMEM((CNODE,), jnp.float32),
            pltpu.VMEM((64,), jnp.float32),
            pltpu.VMEM((16,), jnp.float32),
            pltpu.VMEM((16,), jnp.float32),
            pltpu.VMEM((2 * CNODE,), jnp.float32),
            pltpu.VMEM((2 * CNODE,), jnp.float32),
        ],
    )


def _pad_w(w):
    out = jnp.zeros((8, 8), jnp.float32)
    return out.at[: w.shape[0], : w.shape[1]].set(w).reshape(-1)


def _pad_b(b):
    return jnp.zeros((16,), jnp.float32).at[: b.shape[0]].set(b)


def kernel(x, edge_index, W1, b1, W2, b2, W3, b3, W4, b4, W5, b5, W6, b6,
           W7, b7, Wc, bc):
    src = edge_index[0]
    dst = edge_index[1]
    npadidx = (N + (jnp.arange(EPAD - E, dtype=jnp.int32) % (NPAD - N)))
    srcp = jnp.concatenate([src, npadidx])
    dstp = jnp.concatenate([dst, npadidx])
    idx3 = jnp.stack([srcp.reshape(EPAD // 128, 128),
                      dstp.reshape(EPAD // 128, 128)], axis=1)

    w1p = jnp.pad(W1, ((0, 0), (0, 8 - W1.shape[1])))
    h1full = _xw1(x, w1p)
    h1flat = h1full[:NPAD].reshape(-1)

    degp = _make_deg()(idx3)
    dis, g = _make_disg1()(degp, h1flat)

    ws = [W2, W3, W4, W5, W6, W7]
    bs = [b1, b2, b3, b4, b5, b6]
    for l in range(6):
        d_in, d_out = DIMS[l], DIMS[l + 1]
        accp = _make_edge(d_in)(idx3, g)
        g = _make_node(d_in, d_out, True)(accp, g, dis, _pad_w(ws[l]),
                                          _pad_b(bs[l]))
    accp = _make_edge(DIMS[6])(idx3, g)
    out_f, h7_f = _make_final()(accp, g, dis, _pad_w(Wc), _pad_b(b7),
                                _pad_b(bc))
    out = out_f.reshape(NPAD, 2)[:N]
    h7 = h7_f.reshape(NPAD, 2)[:N]
    return (out, h7)
